# baseline (speedup 1.0000x reference)
import functools

import jax
import jax.numpy as jnp
import numpy as np
from jax import lax
from jax.experimental import pallas as pl
from jax.experimental.pallas import tpu as pltpu
from jax.experimental.pallas import tpu_sc as plsc

V = 1000000
D = 64
EXP = 32
B = 4096
L = 200
BL = B * L

NC = 2
NS = 16
NW = NC * NS
PER_W = BL // NW
CHUNK = 128
N_CHUNKS = PER_W // CHUNK


def _pe_np(seq_len, d_model):
    pos = np.arange(seq_len, dtype=np.float32)[:, None]
    div = np.exp(np.arange(0, d_model, 2, dtype=np.float32) * -(np.log(10000.0) / d_model))
    pe = np.zeros((seq_len, d_model), dtype=np.float32)
    pe[:, 0::2] = np.sin(pos * div)
    pe[:, 1::2] = np.cos(pos * div)
    return pe


def _sc_gather_body(idx_hbm, table_hbm, out_hbm, idx_v, rows_v, sem):
    wid = lax.axis_index("s") * NC + lax.axis_index("c")
    base = wid * PER_W

    def body(i, carry):
        off = base + i * CHUNK
        pltpu.sync_copy(idx_hbm.at[pl.ds(off, CHUNK)], idx_v)
        pltpu.async_copy(table_hbm.at[idx_v], rows_v, sem).wait()
        pltpu.sync_copy(rows_v, out_hbm.at[pl.ds(off, CHUNK)])
        return carry

    lax.fori_loop(0, N_CHUNKS, body, 0, unroll=False)


@jax.jit
def _sc_gather(idx, table):
    fn = functools.partial(
        pl.kernel,
        mesh=plsc.VectorSubcoreMesh(core_axis_name="c", subcore_axis_name="s"),
        compiler_params=pltpu.CompilerParams(use_tc_tiling_on_sc=False),
        out_type=jax.ShapeDtypeStruct((BL, D), jnp.float32),
        scratch_types=[
            pltpu.VMEM((CHUNK,), jnp.int32),
            pltpu.VMEM((CHUNK, D), jnp.float32),
            pltpu.SemaphoreType.DMA,
        ],
    )(_sc_gather_body)
    return fn(idx, table)


BB = 64


def _tc_fuse_body(tok_ref, time_ref, pe_ref, fr_ref, ph_ref, w_ref, out_ref):
    t = time_ref[...]
    h = jnp.cos(t[..., None] * fr_ref[0][None, None, :]
                + ph_ref[0][None, None, :])
    te = lax.dot_general(
        h.reshape(BB * L, EXP), w_ref[...],
        (((1,), (0,)), ((), ())),
        preferred_element_type=jnp.float32,
        precision=lax.Precision.HIGHEST,
    ).reshape(BB, L, D)
    out_ref[...] = tok_ref[...] + pe_ref[...][None] + te


@jax.jit
def _tc_fuse(tok_emb, time_seq, pe2, fr, ph, w):
    grid = (B // BB,)
    return pl.pallas_call(
        _tc_fuse_body,
        grid=grid,
        in_specs=[
            pl.BlockSpec((BB, L, D), lambda i: (i, 0, 0)),
            pl.BlockSpec((BB, L), lambda i: (i, 0)),
            pl.BlockSpec((L, D), lambda i: (0, 0)),
            pl.BlockSpec((1, EXP), lambda i: (0, 0)),
            pl.BlockSpec((1, EXP), lambda i: (0, 0)),
            pl.BlockSpec((EXP, D), lambda i: (0, 0)),
        ],
        out_specs=pl.BlockSpec((BB, L, D), lambda i: (i, 0, 0)),
        out_shape=jax.ShapeDtypeStruct((B, L, D), jnp.float32),
    )(tok_emb, time_seq, pe2, fr, ph, w)


def kernel(token_seq, time_seq, token_table, basis_freq, phase, W_time, b_time):
    idx = token_seq.reshape(-1).astype(jnp.int32)
    tok_emb = _sc_gather(idx, token_table).reshape(B, L, D)
    pe2 = jnp.asarray(_pe_np(L, D)) + b_time[None, :]
    return _tc_fuse(tok_emb, time_seq, pe2,
                    basis_freq.reshape(1, EXP), phase.reshape(1, EXP), W_time)


# speedup vs baseline: 1.9464x; 1.9464x over previous
import functools

import jax
import jax.numpy as jnp
import numpy as np
from jax import lax
from jax.experimental import pallas as pl
from jax.experimental.pallas import tpu as pltpu
from jax.experimental.pallas import tpu_sc as plsc

V = 1000000
D = 64
EXP = 32
B = 4096
L = 200

NC = 2
NS = 16
NW = NC * NS
ROWS_PER_W = B // NW
SPLIT_A = 128
SPLIT_B = L - SPLIT_A


def _pe_np(seq_len, d_model):
    pos = np.arange(seq_len, dtype=np.float32)[:, None]
    div = np.exp(np.arange(0, d_model, 2, dtype=np.float32) * -(np.log(10000.0) / d_model))
    pe = np.zeros((seq_len, d_model), dtype=np.float32)
    pe[:, 0::2] = np.sin(pos * div)
    pe[:, 1::2] = np.cos(pos * div)
    return pe


def _sc_gather_body(tok_hbm, table_hbm, out_hbm, idx_v, rows_v, gsem, ssem):
    wid = lax.axis_index("s") * NC + lax.axis_index("c")
    base = wid * ROWS_PER_W

    def fire_gather(slot, r):
        pltpu.sync_copy(tok_hbm.at[r], idx_v.at[slot])
        pltpu.async_copy(table_hbm.at[idx_v.at[slot, pl.ds(0, SPLIT_A)]],
                         rows_v.at[slot, pl.ds(0, SPLIT_A)], gsem.at[slot])
        pltpu.async_copy(table_hbm.at[idx_v.at[slot, pl.ds(SPLIT_A, SPLIT_B)]],
                         rows_v.at[slot, pl.ds(SPLIT_A, SPLIT_B)], gsem.at[slot])

    def wait_gather(slot):
        pltpu.make_async_copy(table_hbm.at[idx_v.at[slot, pl.ds(0, SPLIT_A)]],
                              rows_v.at[slot, pl.ds(0, SPLIT_A)], gsem.at[slot]).wait()
        pltpu.make_async_copy(table_hbm.at[idx_v.at[slot, pl.ds(SPLIT_A, SPLIT_B)]],
                              rows_v.at[slot, pl.ds(SPLIT_A, SPLIT_B)], gsem.at[slot]).wait()

    def wait_scatter(slot, r):
        pltpu.make_async_copy(rows_v.at[slot], out_hbm.at[r], ssem.at[slot]).wait()

    fire_gather(0, base)

    def body(i, carry):
        par = lax.rem(i, 2)
        nxt = 1 - par

        @pl.when(i + 1 < ROWS_PER_W)
        def _():
            @pl.when(i >= 1)
            def _():
                wait_scatter(nxt, base + i - 1)
            fire_gather(nxt, base + i + 1)

        wait_gather(par)
        pltpu.async_copy(rows_v.at[par], out_hbm.at[base + i], ssem.at[par])
        return carry

    lax.fori_loop(0, ROWS_PER_W, body, 0, unroll=False)
    wait_scatter(lax.rem(ROWS_PER_W - 1, 2), base + ROWS_PER_W - 1)


@jax.jit
def _sc_gather(tok, table):
    fn = functools.partial(
        pl.kernel,
        mesh=plsc.VectorSubcoreMesh(core_axis_name="c", subcore_axis_name="s"),
        compiler_params=pltpu.CompilerParams(use_tc_tiling_on_sc=False),
        out_type=jax.ShapeDtypeStruct((B, L, D), jnp.float32),
        scratch_types=[
            pltpu.VMEM((2, L), jnp.int32),
            pltpu.VMEM((2, L, D), jnp.float32),
            pltpu.SemaphoreType.DMA((2,)),
            pltpu.SemaphoreType.DMA((2,)),
        ],
    )(_sc_gather_body)
    return fn(tok, table)


BB = 64

_COS_C0 = 0.999999443678766
_COS_C1 = -0.49999558165578417
_COS_C2 = 0.04166103279005172
_COS_C3 = -0.001386274731578642
_COS_C4 = 2.425319249599542e-05
_COS_C5 = -2.2193949944101022e-07
_TWO_PI_INV = 0.15915494309189535
_RED_HI = 6.28125
_RED_LO = 0.0019353071795864769
_MAGIC = 12582912.0


def _fast_cos(x):
    k = (x * _TWO_PI_INV + _MAGIC) - _MAGIC
    r = (x - k * _RED_HI) - k * _RED_LO
    y = r * r
    p = _COS_C5
    p = p * y + _COS_C4
    p = p * y + _COS_C3
    p = p * y + _COS_C2
    p = p * y + _COS_C1
    return p * y + _COS_C0


def _tc_fuse_body(tok_ref, time_ref, pe_ref, fr_ref, ph_ref, w_ref, out_ref):
    t = time_ref[...]
    h = _fast_cos(t[..., None] * fr_ref[0][None, None, :]
                  + ph_ref[0][None, None, :])
    te = lax.dot_general(
        h.reshape(BB * L, EXP), w_ref[...],
        (((1,), (0,)), ((), ())),
        preferred_element_type=jnp.float32,
    ).reshape(BB, L, D)
    out_ref[...] = tok_ref[...] + pe_ref[...][None] + te


@jax.jit
def _tc_fuse(tok_emb, time_seq, pe2, fr, ph, w):
    grid = (B // BB,)
    return pl.pallas_call(
        _tc_fuse_body,
        grid=grid,
        in_specs=[
            pl.BlockSpec((BB, L, D), lambda i: (i, 0, 0)),
            pl.BlockSpec((BB, L), lambda i: (i, 0)),
            pl.BlockSpec((L, D), lambda i: (0, 0)),
            pl.BlockSpec((1, EXP), lambda i: (0, 0)),
            pl.BlockSpec((1, EXP), lambda i: (0, 0)),
            pl.BlockSpec((EXP, D), lambda i: (0, 0)),
        ],
        out_specs=pl.BlockSpec((BB, L, D), lambda i: (i, 0, 0)),
        out_shape=jax.ShapeDtypeStruct((B, L, D), jnp.float32),
    )(tok_emb, time_seq, pe2, fr, ph, w)


def kernel(token_seq, time_seq, token_table, basis_freq, phase, W_time, b_time):
    tok_emb = _sc_gather(token_seq.astype(jnp.int32), token_table)
    pe2 = jnp.asarray(_pe_np(L, D)) + b_time[None, :]
    return _tc_fuse(tok_emb, time_seq, pe2,
                    basis_freq.reshape(1, EXP), phase.reshape(1, EXP), W_time)


# speedup vs baseline: 1.9641x; 1.0091x over previous
import functools

import jax
import jax.numpy as jnp
import numpy as np
from jax import lax
from jax.experimental import pallas as pl
from jax.experimental.pallas import tpu as pltpu
from jax.experimental.pallas import tpu_sc as plsc

V = 1000000
D = 64
EXP = 32
B = 4096
L = 200

NC = 2
NS = 16
NW = NC * NS
ROWS_PER_W = B // NW
SPLIT_A = 128
SPLIT_B = L - SPLIT_A


def _pe_np(seq_len, d_model):
    pos = np.arange(seq_len, dtype=np.float32)[:, None]
    div = np.exp(np.arange(0, d_model, 2, dtype=np.float32) * -(np.log(10000.0) / d_model))
    pe = np.zeros((seq_len, d_model), dtype=np.float32)
    pe[:, 0::2] = np.sin(pos * div)
    pe[:, 1::2] = np.cos(pos * div)
    return pe


def _sc_gather_body(tok_hbm, table_hbm, out_hbm, idx_all, rows_v, gsem, ssem):
    wid = lax.axis_index("s") * NC + lax.axis_index("c")
    base = wid * ROWS_PER_W

    pltpu.sync_copy(tok_hbm.at[pl.ds(base, ROWS_PER_W)], idx_all)

    def fire_gather(slot, i):
        pltpu.async_copy(table_hbm.at[idx_all.at[i, pl.ds(0, SPLIT_A)]],
                         rows_v.at[slot, pl.ds(0, SPLIT_A)], gsem.at[slot])
        pltpu.async_copy(table_hbm.at[idx_all.at[i, pl.ds(SPLIT_A, SPLIT_B)]],
                         rows_v.at[slot, pl.ds(SPLIT_A, SPLIT_B)], gsem.at[slot])

    def wait_gather(slot, i):
        pltpu.make_async_copy(table_hbm.at[idx_all.at[i, pl.ds(0, SPLIT_A)]],
                              rows_v.at[slot, pl.ds(0, SPLIT_A)], gsem.at[slot]).wait()
        pltpu.make_async_copy(table_hbm.at[idx_all.at[i, pl.ds(SPLIT_A, SPLIT_B)]],
                              rows_v.at[slot, pl.ds(SPLIT_A, SPLIT_B)], gsem.at[slot]).wait()

    def wait_scatter(slot, r):
        pltpu.make_async_copy(rows_v.at[slot], out_hbm.at[r], ssem.at[slot]).wait()

    fire_gather(0, 0)

    def body(i, carry):
        par = lax.rem(i, 2)
        nxt = 1 - par

        @pl.when(i + 1 < ROWS_PER_W)
        def _():
            @pl.when(i >= 1)
            def _():
                wait_scatter(nxt, base + i - 1)
            fire_gather(nxt, i + 1)

        wait_gather(par, i)
        pltpu.async_copy(rows_v.at[par], out_hbm.at[base + i], ssem.at[par])
        return carry

    lax.fori_loop(0, ROWS_PER_W, body, 0, unroll=False)
    wait_scatter(lax.rem(ROWS_PER_W - 1, 2), base + ROWS_PER_W - 1)


@jax.jit
def _sc_gather(tok, table):
    fn = functools.partial(
        pl.kernel,
        mesh=plsc.VectorSubcoreMesh(core_axis_name="c", subcore_axis_name="s"),
        compiler_params=pltpu.CompilerParams(use_tc_tiling_on_sc=False),
        out_type=jax.ShapeDtypeStruct((B, L, D), jnp.float32),
        scratch_types=[
            pltpu.VMEM((ROWS_PER_W, L), jnp.int32),
            pltpu.VMEM((2, L, D), jnp.float32),
            pltpu.SemaphoreType.DMA((2,)),
            pltpu.SemaphoreType.DMA((2,)),
        ],
    )(_sc_gather_body)
    return fn(tok, table)


BB = 64

_COS_C0 = 0.999999443678766
_COS_C1 = -0.49999558165578417
_COS_C2 = 0.04166103279005172
_COS_C3 = -0.001386274731578642
_COS_C4 = 2.425319249599542e-05
_COS_C5 = -2.2193949944101022e-07
_TWO_PI_INV = 0.15915494309189535
_RED_HI = 6.28125
_RED_LO = 0.0019353071795864769
_MAGIC = 12582912.0


def _fast_cos(x):
    k = (x * _TWO_PI_INV + _MAGIC) - _MAGIC
    r = (x - k * _RED_HI) - k * _RED_LO
    y = r * r
    p = _COS_C5
    p = p * y + _COS_C4
    p = p * y + _COS_C3
    p = p * y + _COS_C2
    p = p * y + _COS_C1
    return p * y + _COS_C0


def _tc_fuse_body(tok_ref, time_ref, pe_ref, fr_ref, ph_ref, w_ref, out_ref):
    t = time_ref[...]
    h = _fast_cos(t[..., None] * fr_ref[0][None, None, :]
                  + ph_ref[0][None, None, :])
    te = lax.dot_general(
        h.reshape(BB * L, EXP), w_ref[...],
        (((1,), (0,)), ((), ())),
        preferred_element_type=jnp.float32,
    ).reshape(BB, L, D)
    out_ref[...] = tok_ref[...] + pe_ref[...][None] + te


@jax.jit
def _tc_fuse(tok_emb, time_seq, pe2, fr, ph, w):
    grid = (B // BB,)
    return pl.pallas_call(
        _tc_fuse_body,
        grid=grid,
        in_specs=[
            pl.BlockSpec((BB, L, D), lambda i: (i, 0, 0)),
            pl.BlockSpec((BB, L), lambda i: (i, 0)),
            pl.BlockSpec((L, D), lambda i: (0, 0)),
            pl.BlockSpec((1, EXP), lambda i: (0, 0)),
            pl.BlockSpec((1, EXP), lambda i: (0, 0)),
            pl.BlockSpec((EXP, D), lambda i: (0, 0)),
        ],
        out_specs=pl.BlockSpec((BB, L, D), lambda i: (i, 0, 0)),
        out_shape=jax.ShapeDtypeStruct((B, L, D), jnp.float32),
    )(tok_emb, time_seq, pe2, fr, ph, w)


def kernel(token_seq, time_seq, token_table, basis_freq, phase, W_time, b_time):
    tok_emb = _sc_gather(token_seq.astype(jnp.int32), token_table)
    pe2 = jnp.asarray(_pe_np(L, D)) + b_time[None, :]
    return _tc_fuse(tok_emb, time_seq, pe2,
                    basis_freq.reshape(1, EXP), phase.reshape(1, EXP), W_time)


# speedup vs baseline: 1.9953x; 1.0159x over previous
import functools

import jax
import jax.numpy as jnp
import numpy as np
from jax import lax
from jax.experimental import pallas as pl
from jax.experimental.pallas import tpu as pltpu
from jax.experimental.pallas import tpu_sc as plsc

V = 1000000
D = 64
EXP = 32
B = 4096
L = 200

NC = 2
NS = 16
NW = NC * NS

NSLICE = 4
BSL = B // NSLICE
SL_ROWS = BSL // NW
SPLIT_A = 128
SPLIT_B = L - SPLIT_A


def _pe_np(seq_len, d_model):
    pos = np.arange(seq_len, dtype=np.float32)[:, None]
    div = np.exp(np.arange(0, d_model, 2, dtype=np.float32) * -(np.log(10000.0) / d_model))
    pe = np.zeros((seq_len, d_model), dtype=np.float32)
    pe[:, 0::2] = np.sin(pos * div)
    pe[:, 1::2] = np.cos(pos * div)
    return pe


def _sc_gather_body(base0, tok_hbm, table_hbm, out_hbm, idx_all, rows_v, gsem, ssem):
    wid = lax.axis_index("s") * NC + lax.axis_index("c")
    lbase = wid * SL_ROWS

    pltpu.sync_copy(tok_hbm.at[pl.ds(base0 + lbase, SL_ROWS)], idx_all)

    def fire_gather(slot, i):
        pltpu.async_copy(table_hbm.at[idx_all.at[i, pl.ds(0, SPLIT_A)]],
                         rows_v.at[slot, pl.ds(0, SPLIT_A)], gsem.at[slot])
        pltpu.async_copy(table_hbm.at[idx_all.at[i, pl.ds(SPLIT_A, SPLIT_B)]],
                         rows_v.at[slot, pl.ds(SPLIT_A, SPLIT_B)], gsem.at[slot])

    def wait_gather(slot, i):
        pltpu.make_async_copy(table_hbm.at[idx_all.at[i, pl.ds(0, SPLIT_A)]],
                              rows_v.at[slot, pl.ds(0, SPLIT_A)], gsem.at[slot]).wait()
        pltpu.make_async_copy(table_hbm.at[idx_all.at[i, pl.ds(SPLIT_A, SPLIT_B)]],
                              rows_v.at[slot, pl.ds(SPLIT_A, SPLIT_B)], gsem.at[slot]).wait()

    def wait_scatter(slot, r):
        pltpu.make_async_copy(rows_v.at[slot], out_hbm.at[r], ssem.at[slot]).wait()

    fire_gather(0, 0)

    def body(i, carry):
        par = lax.rem(i, 2)
        nxt = 1 - par

        @pl.when(i + 1 < SL_ROWS)
        def _():
            @pl.when(i >= 1)
            def _():
                wait_scatter(nxt, lbase + i - 1)
            fire_gather(nxt, i + 1)

        wait_gather(par, i)
        pltpu.async_copy(rows_v.at[par], out_hbm.at[lbase + i], ssem.at[par])
        return carry

    lax.fori_loop(0, SL_ROWS, body, 0, unroll=False)
    wait_scatter(lax.rem(SL_ROWS - 2, 2), lbase + SL_ROWS - 2)
    wait_scatter(lax.rem(SL_ROWS - 1, 2), lbase + SL_ROWS - 1)


def _sc_gather_slice(tok, table, k):
    fn = functools.partial(
        pl.kernel,
        mesh=plsc.VectorSubcoreMesh(core_axis_name="c", subcore_axis_name="s"),
        compiler_params=pltpu.CompilerParams(use_tc_tiling_on_sc=False),
        out_type=jax.ShapeDtypeStruct((BSL, L, D), jnp.float32),
        scratch_types=[
            pltpu.VMEM((SL_ROWS, L), jnp.int32),
            pltpu.VMEM((2, L, D), jnp.float32),
            pltpu.SemaphoreType.DMA((2,)),
            pltpu.SemaphoreType.DMA((2,)),
        ],
        name=f"sc_gather_s{k}",
    )(functools.partial(_sc_gather_body, k * BSL))
    return fn(tok, table)


BB = 64

_COS_C0 = 0.999999443678766
_COS_C1 = -0.49999558165578417
_COS_C2 = 0.04166103279005172
_COS_C3 = -0.001386274731578642
_COS_C4 = 2.425319249599542e-05
_COS_C5 = -2.2193949944101022e-07
_TWO_PI_INV = 0.15915494309189535
_RED_HI = 6.28125
_RED_LO = 0.0019353071795864769
_MAGIC = 12582912.0


def _fast_cos(x):
    k = (x * _TWO_PI_INV + _MAGIC) - _MAGIC
    r = (x - k * _RED_HI) - k * _RED_LO
    y = r * r
    p = _COS_C5
    p = p * y + _COS_C4
    p = p * y + _COS_C3
    p = p * y + _COS_C2
    p = p * y + _COS_C1
    return p * y + _COS_C0


def _fuse_math(tok, t, pe, fr, ph, w):
    h = _fast_cos(t[..., None] * fr[None, None, :] + ph[None, None, :])
    te = lax.dot_general(
        h.reshape(BB * L, EXP), w,
        (((1,), (0,)), ((), ())),
        preferred_element_type=jnp.float32,
    ).reshape(BB, L, D)
    return tok + pe[None] + te


def _tc_fuse_body0(tok_ref, time_ref, pe_ref, fr_ref, ph_ref, w_ref, out_ref):
    out_ref[...] = _fuse_math(tok_ref[...], time_ref[...], pe_ref[...],
                              fr_ref[0], ph_ref[0], w_ref[...])


def _tc_fuse_body(prev_ref, tok_ref, time_ref, pe_ref, fr_ref, ph_ref, w_ref, out_ref):
    out_ref[...] = _fuse_math(tok_ref[...], time_ref[...], pe_ref[...],
                              fr_ref[0], ph_ref[0], w_ref[...])


_CONST_SPECS = [
    pl.BlockSpec((L, D), lambda i: (0, 0)),
    pl.BlockSpec((1, EXP), lambda i: (0, 0)),
    pl.BlockSpec((1, EXP), lambda i: (0, 0)),
    pl.BlockSpec((EXP, D), lambda i: (0, 0)),
]


def _tc_fuse_slice(prev, tok_k, time_seq, pe2, fr, ph, w, k):
    gb = BSL // BB
    off = k * gb
    in_specs = [
        pl.BlockSpec((BB, L, D), lambda i: (i, 0, 0)),
        pl.BlockSpec((BB, L), lambda i: (off + i, 0)),
    ] + _CONST_SPECS
    out_spec = pl.BlockSpec((BB, L, D), lambda i: (off + i, 0, 0))
    if prev is None:
        return pl.pallas_call(
            _tc_fuse_body0,
            grid=(gb,),
            in_specs=in_specs,
            out_specs=out_spec,
            out_shape=jax.ShapeDtypeStruct((B, L, D), jnp.float32),
            name=f"tc_fuse_s{k}",
        )(tok_k, time_seq, pe2, fr, ph, w)
    return pl.pallas_call(
        _tc_fuse_body,
        grid=(gb,),
        in_specs=[pl.BlockSpec(memory_space=pl.ANY)] + in_specs,
        out_specs=out_spec,
        out_shape=jax.ShapeDtypeStruct((B, L, D), jnp.float32),
        input_output_aliases={0: 0},
        name=f"tc_fuse_s{k}",
    )(prev, tok_k, time_seq, pe2, fr, ph, w)


def kernel(token_seq, time_seq, token_table, basis_freq, phase, W_time, b_time):
    tok_i32 = token_seq.astype(jnp.int32)
    pe2 = jnp.asarray(_pe_np(L, D)) + b_time[None, :]
    fr = basis_freq.reshape(1, EXP)
    ph = phase.reshape(1, EXP)
    toks = [_sc_gather_slice(tok_i32, token_table, k) for k in range(NSLICE)]
    out = None
    for k in range(NSLICE):
        out = _tc_fuse_slice(out, toks[k], time_seq, pe2, fr, ph, W_time, k)
    return out


# speedup vs baseline: 2.2270x; 1.1162x over previous
import functools

import jax
import jax.numpy as jnp
import numpy as np
from jax import lax
from jax.experimental import pallas as pl
from jax.experimental.pallas import tpu as pltpu
from jax.experimental.pallas import tpu_sc as plsc

V = 1000000
D = 64
EXP = 32
B = 4096
L = 200

NC = 2
NS = 16
NW = NC * NS

NSLICE = 4
BSL = B // NSLICE
SL_ROWS = BSL // NW
SPLIT_A = 128
SPLIT_B = L - SPLIT_A


def _pe_np(seq_len, d_model):
    pos = np.arange(seq_len, dtype=np.float32)[:, None]
    div = np.exp(np.arange(0, d_model, 2, dtype=np.float32) * -(np.log(10000.0) / d_model))
    pe = np.zeros((seq_len, d_model), dtype=np.float32)
    pe[:, 0::2] = np.sin(pos * div)
    pe[:, 1::2] = np.cos(pos * div)
    return pe


def _sc_gather_body(base0, tok_hbm, table_hbm, out_hbm, idx_all, rows_v, gsem, ssem):
    wid = lax.axis_index("s") * NC + lax.axis_index("c")
    lbase = wid * SL_ROWS

    pltpu.sync_copy(tok_hbm.at[pl.ds(base0 + lbase, SL_ROWS)], idx_all)

    def fire_gather(slot, i):
        pltpu.async_copy(table_hbm.at[idx_all.at[i, pl.ds(0, SPLIT_A)]],
                         rows_v.at[slot, pl.ds(0, SPLIT_A)], gsem.at[slot])
        pltpu.async_copy(table_hbm.at[idx_all.at[i, pl.ds(SPLIT_A, SPLIT_B)]],
                         rows_v.at[slot, pl.ds(SPLIT_A, SPLIT_B)], gsem.at[slot])

    def wait_gather(slot, i):
        pltpu.make_async_copy(table_hbm.at[idx_all.at[i, pl.ds(0, SPLIT_A)]],
                              rows_v.at[slot, pl.ds(0, SPLIT_A)], gsem.at[slot]).wait()
        pltpu.make_async_copy(table_hbm.at[idx_all.at[i, pl.ds(SPLIT_A, SPLIT_B)]],
                              rows_v.at[slot, pl.ds(SPLIT_A, SPLIT_B)], gsem.at[slot]).wait()

    def wait_scatter(slot, r):
        pltpu.make_async_copy(rows_v.at[slot], out_hbm.at[r], ssem.at[slot]).wait()

    fire_gather(0, 0)

    def body(i, carry):
        par = lax.rem(i, 2)
        nxt = 1 - par

        @pl.when(i + 1 < SL_ROWS)
        def _():
            @pl.when(i >= 1)
            def _():
                wait_scatter(nxt, lbase + i - 1)
            fire_gather(nxt, i + 1)

        wait_gather(par, i)
        pltpu.async_copy(rows_v.at[par], out_hbm.at[lbase + i], ssem.at[par])
        return carry

    lax.fori_loop(0, SL_ROWS, body, 0, unroll=False)
    wait_scatter(lax.rem(SL_ROWS - 2, 2), lbase + SL_ROWS - 2)
    wait_scatter(lax.rem(SL_ROWS - 1, 2), lbase + SL_ROWS - 1)


def _sc_gather_slice(tok, table, k):
    fn = functools.partial(
        pl.kernel,
        mesh=plsc.VectorSubcoreMesh(core_axis_name="c", subcore_axis_name="s"),
        compiler_params=pltpu.CompilerParams(use_tc_tiling_on_sc=False),
        out_type=jax.ShapeDtypeStruct((BSL, L, D), jnp.float32),
        scratch_types=[
            pltpu.VMEM((SL_ROWS, L), jnp.int32),
            pltpu.VMEM((2, L, D), jnp.float32),
            pltpu.SemaphoreType.DMA((2,)),
            pltpu.SemaphoreType.DMA((2,)),
        ],
        name=f"sc_gather_s{k}",
    )(functools.partial(_sc_gather_body, k * BSL))
    return fn(tok, table)


BB = 64

_COS_C0 = 0.999999443678766
_COS_C1 = -0.49999558165578417
_COS_C2 = 0.04166103279005172
_COS_C3 = -0.001386274731578642
_COS_C4 = 2.425319249599542e-05
_COS_C5 = -2.2193949944101022e-07
_TWO_PI_INV = 0.15915494309189535
_RED_HI = 6.28125
_RED_LO = 0.0019353071795864769
_MAGIC = 12582912.0


def _fast_cos(x):
    k = (x * _TWO_PI_INV + _MAGIC) - _MAGIC
    r = (x - k * _RED_HI) - k * _RED_LO
    y = r * r
    p = _COS_C5
    p = p * y + _COS_C4
    p = p * y + _COS_C3
    p = p * y + _COS_C2
    p = p * y + _COS_C1
    return p * y + _COS_C0


LP = L // 2


def _fuse_math(tok2, te_, tod, pe_pair, fr, ph, w2):
    arg_e = te_[..., None] * fr[None, None, :] + ph[None, None, :]
    arg_o = tod[..., None] * fr[None, None, :] + ph[None, None, :]
    h = _fast_cos(jnp.concatenate([arg_e, arg_o], axis=-1))
    tem = lax.dot_general(
        h.reshape(BB * LP, 2 * EXP), w2,
        (((1,), (0,)), ((), ())),
        preferred_element_type=jnp.float32,
    ).reshape(BB, LP, 2 * D)
    return (tok2.reshape(BB, LP, 2 * D) + pe_pair[None] + tem).reshape(BB * LP, 2 * D)


def _tc_fuse_body0(tok_ref, te_ref, to_ref, pe_ref, fr_ref, ph_ref, w2_ref, out_ref):
    out_ref[...] = _fuse_math(tok_ref[...], te_ref[...], to_ref[...],
                              pe_ref[...], fr_ref[0], ph_ref[0], w2_ref[...])


def _tc_fuse_body(prev_ref, tok_ref, te_ref, to_ref, pe_ref, fr_ref, ph_ref,
                  w2_ref, out_ref):
    out_ref[...] = _fuse_math(tok_ref[...], te_ref[...], to_ref[...],
                              pe_ref[...], fr_ref[0], ph_ref[0], w2_ref[...])


def _tc_fuse_slice(prev, tok_k, t_even, t_odd, pe_pair, fr, ph, w2, k):
    gb = BSL // BB
    off = k * gb
    in_specs = [
        pl.BlockSpec((BB * LP, 2 * D), lambda i: (i, 0)),
        pl.BlockSpec((BB, LP), lambda i: (off + i, 0)),
        pl.BlockSpec((BB, LP), lambda i: (off + i, 0)),
        pl.BlockSpec((LP, 2 * D), lambda i: (0, 0)),
        pl.BlockSpec((1, EXP), lambda i: (0, 0)),
        pl.BlockSpec((1, EXP), lambda i: (0, 0)),
        pl.BlockSpec((2 * EXP, 2 * D), lambda i: (0, 0)),
    ]
    out_spec = pl.BlockSpec((BB * LP, 2 * D), lambda i: (off + i, 0))
    out_shape = jax.ShapeDtypeStruct((B * LP, 2 * D), jnp.float32)
    if prev is None:
        return pl.pallas_call(
            _tc_fuse_body0,
            grid=(gb,),
            in_specs=in_specs,
            out_specs=out_spec,
            out_shape=out_shape,
            name=f"tc_fuse_s{k}",
        )(tok_k, t_even, t_odd, pe_pair, fr, ph, w2)
    return pl.pallas_call(
        _tc_fuse_body,
        grid=(gb,),
        in_specs=[pl.BlockSpec(memory_space=pl.ANY)] + in_specs,
        out_specs=out_spec,
        out_shape=out_shape,
        input_output_aliases={0: 0},
        name=f"tc_fuse_s{k}",
    )(prev, tok_k, t_even, t_odd, pe_pair, fr, ph, w2)


def kernel(token_seq, time_seq, token_table, basis_freq, phase, W_time, b_time):
    tok_i32 = token_seq.astype(jnp.int32)
    pe2 = jnp.asarray(_pe_np(L, D)) + b_time[None, :]
    pe_pair = pe2.reshape(LP, 2 * D)
    fr = basis_freq.reshape(1, EXP)
    ph = phase.reshape(1, EXP)
    w2 = jnp.zeros((2 * EXP, 2 * D), jnp.float32)
    w2 = w2.at[:EXP, :D].set(W_time).at[EXP:, D:].set(W_time)
    t_even = time_seq[:, 0::2]
    t_odd = time_seq[:, 1::2]
    toks = [_sc_gather_slice(tok_i32, token_table, k).reshape(BSL * LP, 2 * D)
            for k in range(NSLICE)]
    out = None
    for k in range(NSLICE):
        out = _tc_fuse_slice(out, toks[k], t_even, t_odd, pe_pair, fr, ph, w2, k)
    return out.reshape(B, L, D)
